# baseline (speedup 1.0000x reference)
import functools

import jax
import jax.numpy as jnp
from jax import lax
from jax.experimental import pallas as pl
from jax.experimental.pallas import tpu as pltpu
from jax.experimental.pallas import tpu_sc as plsc

N_NODES = 10000
N_EDGES = 320000
D_HID = 128
D3 = 16

NUM_CORES = 2
NUM_SUBCORES = 16
NW = NUM_CORES * NUM_SUBCORES
SLABW = 128
NROW = -(-N_EDGES // (NW * SLABW))
NROW = -(-NROW // 8) * 8
E_PAD = NW * NROW * SLABW
ROWS_PAD = 10240
STRIPE = ROWS_PAD // NUM_SUBCORES
TRASH = ROWS_PAD - 1


def _make_sc_agg(D):
  mesh = plsc.VectorSubcoreMesh(core_axis_name="c", subcore_axis_name="s")

  @functools.partial(
      pl.kernel,
      mesh=mesh,
      out_type=(
          jax.ShapeDtypeStruct((ROWS_PAD, D), jnp.float32),
          jax.ShapeDtypeStruct((ROWS_PAD, D), jnp.float32),
      ),
      scratch_types=[
          pltpu.VMEM_SHARED((ROWS_PAD, D), jnp.float32),
          pltpu.VMEM((NROW, SLABW), jnp.int32),
          pltpu.VMEM((8, 16, D), jnp.float32),
          pltpu.VMEM((16, D), jnp.float32),
      ] + [pltpu.SemaphoreType.DMA] * 16,
  )
  def agg(y_hbm, edges_hbm, out0, out1, acc, eslab, gb, zb, *sems):
    cid = lax.axis_index("c")
    sid = lax.axis_index("s")
    w = cid * NUM_SUBCORES + sid
    gsem = sems[:8]
    ssem = sems[8:]

    zero = jnp.zeros((16,), jnp.float32)
    for r in range(16):
      for c in range(D // 16):
        zb[r, pl.ds(c * 16, 16)] = zero

    def zero_body(i, carry):
      pltpu.sync_copy(zb, acc.at[pl.ds(sid * STRIPE + i * 16, 16)])
      return carry

    lax.fori_loop(0, STRIPE // 16, zero_body, 0)

    pltpu.sync_copy(edges_hbm.at[w], eslab)

    plsc.subcore_barrier()

    idle = lax.iota(jnp.int32, 16)

    def src_of(j, u):
      return lax.shift_right_logical(eslab[j, pl.ds(u * 16, 16)], 16)

    def dst_of(j, u):
      return jnp.bitwise_and(eslab[j, pl.ds(u * 16, 16)], 0xFFFF)

    def g_start(j, u):
      pltpu.make_async_copy(y_hbm.at[src_of(j, u)], gb.at[u], gsem[u]).start()

    def g_wait(u):
      pltpu.make_async_copy(y_hbm.at[idle], gb.at[u], gsem[u]).wait()

    def s_start(j, u):
      pltpu.async_copy(gb.at[u], acc.at[dst_of(j, u)], ssem[u], add=True)

    def s_wait(u):
      pltpu.make_async_copy(gb.at[u], acc.at[idle], ssem[u]).wait()

    for u in range(8):
      g_start(0, u)
      if u >= 4:
        g_wait(u - 4)
        s_start(0, u - 4)

    def body(j, carry):
      for u in range(8):
        s_wait(u)
        g_start(j, u)
        if u >= 4:
          g_wait(u - 4)
          s_start(j, u - 4)
        else:
          g_wait(u + 4)
          s_start(j - 1, u + 4)
      return carry

    lax.fori_loop(1, NROW, body, 0)

    for u in range(4):
      g_wait(u + 4)
      s_start(NROW - 1, u + 4)
    for u in range(8):
      s_wait(u)

    plsc.subcore_barrier()

    @pl.when(cid == 0)
    def _():
      pltpu.sync_copy(acc.at[pl.ds(sid * STRIPE, STRIPE)],
                      out0.at[pl.ds(sid * STRIPE, STRIPE)])

    @pl.when(cid == 1)
    def _():
      pltpu.sync_copy(acc.at[pl.ds(sid * STRIPE, STRIPE)],
                      out1.at[pl.ds(sid * STRIPE, STRIPE)])

  return agg


_AGG128 = _make_sc_agg(D_HID)

BLK = 1000


def _mm_body(x_ref, w_ref, o_ref):
  o_ref[...] = jnp.dot(x_ref[...], w_ref[...],
                       preferred_element_type=jnp.float32)


def _tc_matmul(x, wt):
  n, k = x.shape
  d_out = wt.shape[1]
  return pl.pallas_call(
      _mm_body,
      grid=(n // BLK,),
      in_specs=[
          pl.BlockSpec((BLK, k), lambda i: (i, 0)),
          pl.BlockSpec((k, d_out), lambda i: (0, 0)),
      ],
      out_specs=pl.BlockSpec((BLK, d_out), lambda i: (i, 0)),
      out_shape=jax.ShapeDtypeStruct((n, d_out), jnp.float32),
  )(x, wt)


def _fuse_body(y_ref, p0_ref, p1_ref, b_ref, w_ref, o_ref):
  h = y_ref[...] + p0_ref[...] + p1_ref[...] + b_ref[...]
  h = jnp.maximum(h, 0.0)
  o_ref[...] = jnp.dot(h, w_ref[...], preferred_element_type=jnp.float32)


def _tc_fuse_matmul(y, p0, p1, b2d, wt):
  n, d = y.shape
  d_out = wt.shape[1]
  return pl.pallas_call(
      _fuse_body,
      grid=(n // BLK,),
      in_specs=[
          pl.BlockSpec((BLK, d), lambda i: (i, 0)),
          pl.BlockSpec((BLK, d), lambda i: (i, 0)),
          pl.BlockSpec((BLK, d), lambda i: (i, 0)),
          pl.BlockSpec((1, d), lambda i: (0, 0)),
          pl.BlockSpec((d, d_out), lambda i: (0, 0)),
      ],
      out_specs=pl.BlockSpec((BLK, d_out), lambda i: (i, 0)),
      out_shape=jax.ShapeDtypeStruct((n, d_out), jnp.float32),
  )(y, p0, p1, b2d, wt)


def _relu_body(y_ref, p0_ref, p1_ref, b_ref, o_ref):
  o_ref[...] = jnp.maximum(
      y_ref[...] + p0_ref[...] + p1_ref[...] + b_ref[...], 0.0)


def _tc_fuse_relu(y, p0, p1, b2d):
  n, d = y.shape
  return pl.pallas_call(
      _relu_body,
      grid=(n // BLK,),
      in_specs=[
          pl.BlockSpec((BLK, d), lambda i: (i, 0)),
          pl.BlockSpec((BLK, d), lambda i: (i, 0)),
          pl.BlockSpec((BLK, d), lambda i: (i, 0)),
          pl.BlockSpec((1, d), lambda i: (0, 0)),
      ],
      out_specs=pl.BlockSpec((BLK, d), lambda i: (i, 0)),
      out_shape=jax.ShapeDtypeStruct((n, d), jnp.float32),
  )(y, p0, p1, b2d)


def _final_body(h_ref, p0_ref, p1_ref, w_ref, b_ref, o_ref):
  h = h_ref[...] + p0_ref[...] + p1_ref[...]
  o_ref[...] = jnp.dot(h, w_ref[...],
                       preferred_element_type=jnp.float32) + b_ref[...]


def _tc_final_matmul(h, p0, p1, wt, b2d):
  n, d = h.shape
  d_out = wt.shape[1]
  return pl.pallas_call(
      _final_body,
      grid=(n // BLK,),
      in_specs=[
          pl.BlockSpec((BLK, d), lambda i: (i, 0)),
          pl.BlockSpec((BLK, d), lambda i: (i, 0)),
          pl.BlockSpec((BLK, d), lambda i: (i, 0)),
          pl.BlockSpec((d, d_out), lambda i: (0, 0)),
          pl.BlockSpec((1, d_out), lambda i: (0, 0)),
      ],
      out_specs=pl.BlockSpec((BLK, d_out), lambda i: (i, 0)),
      out_shape=jax.ShapeDtypeStruct((n, d_out), jnp.float32),
  )(h, p0, p1, wt, b2d)


def kernel(x, edge_index, W1, b1, W2, b2, W3, b3):
  src = edge_index[0].astype(jnp.int32)
  dst = edge_index[1].astype(jnp.int32)
  pad = E_PAD - N_EDGES
  packed = jnp.sort(jnp.bitwise_or(jnp.left_shift(src, 16), dst))
  packed = jnp.concatenate([packed, jnp.full((pad,), TRASH, jnp.int32)])
  edges_p = (packed.reshape(NROW * 8, NW, 16).swapaxes(0, 1)
             .reshape(NW, NROW, SLABW))

  w3p = jnp.zeros((D3, D_HID), jnp.float32).at[:2].set(W3)
  b3p = jnp.zeros((1, D3), jnp.float32).at[0, :2].set(b3)

  y1 = _tc_matmul(x, W1.T)
  p1a, p1b = _AGG128(y1, edges_p)
  y2 = _tc_fuse_matmul(y1, p1a[:N_NODES], p1b[:N_NODES], b1.reshape(1, -1),
                       W2.T)
  p2a, p2b = _AGG128(y2, edges_p)
  h2 = _tc_fuse_relu(y2, p2a[:N_NODES], p2b[:N_NODES], b2.reshape(1, -1))
  p3a, p3b = _AGG128(h2, edges_p)
  out = _tc_final_matmul(h2, p3a[:N_NODES], p3b[:N_NODES], w3p.T, b3p)
  return out[:, :2]


# speedup vs baseline: 1.1079x; 1.1079x over previous
import functools

import jax
import jax.numpy as jnp
from jax import lax
from jax.experimental import pallas as pl
from jax.experimental.pallas import tpu as pltpu
from jax.experimental.pallas import tpu_sc as plsc

N_NODES = 10000
N_EDGES = 320000
D_HID = 128
D3 = 16

NUM_CORES = 2
NUM_SUBCORES = 16
NW = NUM_CORES * NUM_SUBCORES
SLABW = 128
NROW = -(-N_EDGES // (NW * SLABW))
NROW = -(-NROW // 8) * 8
E_PAD = NW * NROW * SLABW
ROWS_PAD = 10240
STRIPE = ROWS_PAD // NUM_SUBCORES
TRASH = ROWS_PAD - 1


def _make_sc_agg(D):
  mesh = plsc.VectorSubcoreMesh(core_axis_name="c", subcore_axis_name="s")

  @functools.partial(
      pl.kernel,
      mesh=mesh,
      out_type=(
          jax.ShapeDtypeStruct((ROWS_PAD, D), jnp.float32),
          jax.ShapeDtypeStruct((ROWS_PAD, D), jnp.float32),
      ),
      scratch_types=[
          pltpu.VMEM_SHARED((ROWS_PAD, D), jnp.float32),
          pltpu.VMEM((NROW, SLABW), jnp.int32),
          pltpu.VMEM((8, 16, D), jnp.float32),
          pltpu.VMEM((16, D), jnp.float32),
      ] + [pltpu.SemaphoreType.DMA] * 16,
  )
  def agg(y_hbm, edges_hbm, out0, out1, acc, eslab, gb, zb, *sems):
    cid = lax.axis_index("c")
    sid = lax.axis_index("s")
    w = cid * NUM_SUBCORES + sid
    gsem = sems[:8]
    ssem = sems[8:]

    zero = jnp.zeros((16,), jnp.float32)
    for r in range(16):
      for c in range(D // 16):
        zb[r, pl.ds(c * 16, 16)] = zero

    def zero_body(i, carry):
      pltpu.sync_copy(zb, acc.at[pl.ds(sid * STRIPE + i * 16, 16)])
      return carry

    lax.fori_loop(0, STRIPE // 16, zero_body, 0)

    pltpu.sync_copy(edges_hbm.at[w], eslab)

    plsc.subcore_barrier()

    idle = lax.iota(jnp.int32, 16)

    def src_of(j, u):
      return lax.shift_right_logical(eslab[j, pl.ds(u * 16, 16)], 16)

    def dst_of(j, u):
      return jnp.bitwise_and(eslab[j, pl.ds(u * 16, 16)], 0xFFFF)

    def g_start(j, u):
      pltpu.make_async_copy(y_hbm.at[src_of(j, u)], gb.at[u], gsem[u]).start()

    def g_wait(u):
      pltpu.make_async_copy(y_hbm.at[idle], gb.at[u], gsem[u]).wait()

    def s_start(j, u):
      pltpu.async_copy(gb.at[u], acc.at[dst_of(j, u)], ssem[u], add=True)

    def s_wait(u):
      pltpu.make_async_copy(gb.at[u], acc.at[idle], ssem[u]).wait()

    for u in range(8):
      g_start(0, u)
      if u >= 1:
        g_wait(u - 1)
        s_start(0, u - 1)

    def body(j, carry):
      for u in range(8):
        s_wait(u)
        g_start(j, u)
        if u == 0:
          g_wait(7)
          s_start(j - 1, 7)
        else:
          g_wait(u - 1)
          s_start(j, u - 1)
      return carry

    lax.fori_loop(1, NROW, body, 0)

    g_wait(7)
    s_start(NROW - 1, 7)
    for u in range(8):
      s_wait(u)

    plsc.subcore_barrier()

    @pl.when(cid == 0)
    def _():
      pltpu.sync_copy(acc.at[pl.ds(sid * STRIPE, STRIPE)],
                      out0.at[pl.ds(sid * STRIPE, STRIPE)])

    @pl.when(cid == 1)
    def _():
      pltpu.sync_copy(acc.at[pl.ds(sid * STRIPE, STRIPE)],
                      out1.at[pl.ds(sid * STRIPE, STRIPE)])

  return agg


_AGG128 = _make_sc_agg(D_HID)

BLK = 1000


def _mm_body(x_ref, w_ref, o_ref):
  o_ref[...] = jnp.dot(x_ref[...], w_ref[...],
                       preferred_element_type=jnp.float32)


def _tc_matmul(x, wt):
  n, k = x.shape
  d_out = wt.shape[1]
  return pl.pallas_call(
      _mm_body,
      grid=(n // BLK,),
      in_specs=[
          pl.BlockSpec((BLK, k), lambda i: (i, 0)),
          pl.BlockSpec((k, d_out), lambda i: (0, 0)),
      ],
      out_specs=pl.BlockSpec((BLK, d_out), lambda i: (i, 0)),
      out_shape=jax.ShapeDtypeStruct((n, d_out), jnp.float32),
  )(x, wt)


def _fuse_body(y_ref, p0_ref, p1_ref, b_ref, w_ref, o_ref):
  h = y_ref[...] + p0_ref[...] + p1_ref[...] + b_ref[...]
  h = jnp.maximum(h, 0.0)
  o_ref[...] = jnp.dot(h, w_ref[...], preferred_element_type=jnp.float32)


def _tc_fuse_matmul(y, p0, p1, b2d, wt):
  n, d = y.shape
  d_out = wt.shape[1]
  return pl.pallas_call(
      _fuse_body,
      grid=(n // BLK,),
      in_specs=[
          pl.BlockSpec((BLK, d), lambda i: (i, 0)),
          pl.BlockSpec((BLK, d), lambda i: (i, 0)),
          pl.BlockSpec((BLK, d), lambda i: (i, 0)),
          pl.BlockSpec((1, d), lambda i: (0, 0)),
          pl.BlockSpec((d, d_out), lambda i: (0, 0)),
      ],
      out_specs=pl.BlockSpec((BLK, d_out), lambda i: (i, 0)),
      out_shape=jax.ShapeDtypeStruct((n, d_out), jnp.float32),
  )(y, p0, p1, b2d, wt)


def _relu_body(y_ref, p0_ref, p1_ref, b_ref, o_ref):
  o_ref[...] = jnp.maximum(
      y_ref[...] + p0_ref[...] + p1_ref[...] + b_ref[...], 0.0)


def _tc_fuse_relu(y, p0, p1, b2d):
  n, d = y.shape
  return pl.pallas_call(
      _relu_body,
      grid=(n // BLK,),
      in_specs=[
          pl.BlockSpec((BLK, d), lambda i: (i, 0)),
          pl.BlockSpec((BLK, d), lambda i: (i, 0)),
          pl.BlockSpec((BLK, d), lambda i: (i, 0)),
          pl.BlockSpec((1, d), lambda i: (0, 0)),
      ],
      out_specs=pl.BlockSpec((BLK, d), lambda i: (i, 0)),
      out_shape=jax.ShapeDtypeStruct((n, d), jnp.float32),
  )(y, p0, p1, b2d)


def _final_body(h_ref, p0_ref, p1_ref, w_ref, b_ref, o_ref):
  h = h_ref[...] + p0_ref[...] + p1_ref[...]
  o_ref[...] = jnp.dot(h, w_ref[...],
                       preferred_element_type=jnp.float32) + b_ref[...]


def _tc_final_matmul(h, p0, p1, wt, b2d):
  n, d = h.shape
  d_out = wt.shape[1]
  return pl.pallas_call(
      _final_body,
      grid=(n // BLK,),
      in_specs=[
          pl.BlockSpec((BLK, d), lambda i: (i, 0)),
          pl.BlockSpec((BLK, d), lambda i: (i, 0)),
          pl.BlockSpec((BLK, d), lambda i: (i, 0)),
          pl.BlockSpec((d, d_out), lambda i: (0, 0)),
          pl.BlockSpec((1, d_out), lambda i: (0, 0)),
      ],
      out_specs=pl.BlockSpec((BLK, d_out), lambda i: (i, 0)),
      out_shape=jax.ShapeDtypeStruct((n, d_out), jnp.float32),
  )(h, p0, p1, wt, b2d)


def kernel(x, edge_index, W1, b1, W2, b2, W3, b3):
  src = edge_index[0].astype(jnp.int32)
  dst = edge_index[1].astype(jnp.int32)
  pad = E_PAD - N_EDGES
  packed = jnp.bitwise_or(jnp.left_shift(src, 16), dst)
  packed = jnp.concatenate([packed, jnp.full((pad,), TRASH, jnp.int32)])
  edges_p = packed.reshape(NW, NROW, SLABW)

  w3p = jnp.zeros((D3, D_HID), jnp.float32).at[:2].set(W3)
  b3p = jnp.zeros((1, D3), jnp.float32).at[0, :2].set(b3)

  y1 = _tc_matmul(x, W1.T)
  p1a, p1b = _AGG128(y1, edges_p)
  y2 = _tc_fuse_matmul(y1, p1a[:N_NODES], p1b[:N_NODES], b1.reshape(1, -1),
                       W2.T)
  p2a, p2b = _AGG128(y2, edges_p)
  h2 = _tc_fuse_relu(y2, p2a[:N_NODES], p2b[:N_NODES], b2.reshape(1, -1))
  p3a, p3b = _AGG128(h2, edges_p)
  out = _tc_final_matmul(h2, p3a[:N_NODES], p3b[:N_NODES], w3p.T, b3p)
  return out[:, :2]


# speedup vs baseline: 1.1323x; 1.0220x over previous
import functools

import jax
import jax.numpy as jnp
from jax import lax
from jax.experimental import pallas as pl
from jax.experimental.pallas import tpu as pltpu
from jax.experimental.pallas import tpu_sc as plsc

N_NODES = 10000
N_EDGES = 320000
D_HID = 128
D3 = 16

NUM_CORES = 2
NUM_SUBCORES = 16
NW = NUM_CORES * NUM_SUBCORES
SLABW = 128
NROW = -(-N_EDGES // (NW * SLABW))
NROW = -(-NROW // 8) * 8
E_PAD = NW * NROW * SLABW
ROWS_PAD = 10240
STRIPE = ROWS_PAD // NUM_SUBCORES
TRASH = ROWS_PAD - 1


def _make_sc_agg(D):
  mesh = plsc.VectorSubcoreMesh(core_axis_name="c", subcore_axis_name="s")

  @functools.partial(
      pl.kernel,
      mesh=mesh,
      out_type=(
          jax.ShapeDtypeStruct((ROWS_PAD, D), jnp.float32),
          jax.ShapeDtypeStruct((ROWS_PAD, D), jnp.float32),
      ),
      scratch_types=[
          pltpu.VMEM_SHARED((ROWS_PAD, D), jnp.float32),
          pltpu.VMEM((NROW + 1, SLABW), jnp.int32),
          pltpu.VMEM((16, D), jnp.float32),
          pltpu.VMEM((16, D), jnp.float32),
          pltpu.VMEM((16, D), jnp.float32),
          pltpu.SemaphoreType.DMA,
          pltpu.SemaphoreType.DMA,
      ],
  )
  def agg(y_hbm, edges_hbm, out0, out1, acc, eslab, g0, g1, zb, sem0, sem1):
    cid = lax.axis_index("c")
    sid = lax.axis_index("s")
    w = cid * NUM_SUBCORES + sid

    zero = jnp.zeros((16,), jnp.float32)
    for r in range(16):
      for c in range(D // 16):
        zb[r, pl.ds(c * 16, 16)] = zero

    def zero_body(i, carry):
      pltpu.sync_copy(zb, acc.at[pl.ds(sid * STRIPE + i * 16, 16)])
      return carry

    lax.fori_loop(0, STRIPE // 16, zero_body, 0)

    pltpu.sync_copy(edges_hbm.at[w], eslab)

    plsc.subcore_barrier()

    bufs = (g0, g1)
    sems = (sem0, sem1)

    def src_of(j, u):
      return lax.shift_right_logical(eslab[j, pl.ds(u * 16, 16)], 16)

    def dst_of(j, u):
      return jnp.bitwise_and(eslab[j, pl.ds(u * 16, 16)], 0xFFFF)

    def gather_start(sv, buf, sem):
      pltpu.make_async_copy(y_hbm.at[sv], buf, sem).start()

    def gather_wait(sv, buf, sem):
      pltpu.make_async_copy(y_hbm.at[sv], buf, sem).wait()

    gather_start(src_of(0, 0), g0, sem0)

    def body(j, carry):
      for k in range(8):
        cur, nxt = bufs[k % 2], bufs[(k + 1) % 2]
        scur, snxt = sems[k % 2], sems[(k + 1) % 2]
        if k < 7:
          sv_n = src_of(j, k + 1)
        else:
          sv_n = src_of(j + 1, 0)
        gather_start(sv_n, nxt, snxt)
        gather_wait(src_of(j, k), cur, scur)
        pltpu.sync_copy(cur, acc.at[dst_of(j, k)], add=True)
      return carry

    lax.fori_loop(0, NROW, body, 0)

    gather_wait(src_of(NROW, 0), g0, sem0)

    plsc.subcore_barrier()

    @pl.when(cid == 0)
    def _():
      pltpu.sync_copy(acc.at[pl.ds(sid * STRIPE, STRIPE)],
                      out0.at[pl.ds(sid * STRIPE, STRIPE)])

    @pl.when(cid == 1)
    def _():
      pltpu.sync_copy(acc.at[pl.ds(sid * STRIPE, STRIPE)],
                      out1.at[pl.ds(sid * STRIPE, STRIPE)])

  return agg


_AGG128 = _make_sc_agg(D_HID)

BLK = 1000


def _mm_body(x_ref, w_ref, o_ref):
  o_ref[...] = jnp.dot(x_ref[...], w_ref[...],
                       preferred_element_type=jnp.float32)


def _tc_matmul(x, wt):
  n, k = x.shape
  d_out = wt.shape[1]
  return pl.pallas_call(
      _mm_body,
      grid=(n // BLK,),
      in_specs=[
          pl.BlockSpec((BLK, k), lambda i: (i, 0)),
          pl.BlockSpec((k, d_out), lambda i: (0, 0)),
      ],
      out_specs=pl.BlockSpec((BLK, d_out), lambda i: (i, 0)),
      out_shape=jax.ShapeDtypeStruct((n, d_out), jnp.float32),
  )(x, wt)


def _fuse_body(y_ref, p0_ref, p1_ref, b_ref, w_ref, o_ref):
  h = y_ref[...] + p0_ref[...] + p1_ref[...] + b_ref[...]
  h = jnp.maximum(h, 0.0)
  o_ref[...] = jnp.dot(h, w_ref[...], preferred_element_type=jnp.float32)


def _tc_fuse_matmul(y, p0, p1, b2d, wt):
  n, d = y.shape
  d_out = wt.shape[1]
  return pl.pallas_call(
      _fuse_body,
      grid=(n // BLK,),
      in_specs=[
          pl.BlockSpec((BLK, d), lambda i: (i, 0)),
          pl.BlockSpec((BLK, d), lambda i: (i, 0)),
          pl.BlockSpec((BLK, d), lambda i: (i, 0)),
          pl.BlockSpec((1, d), lambda i: (0, 0)),
          pl.BlockSpec((d, d_out), lambda i: (0, 0)),
      ],
      out_specs=pl.BlockSpec((BLK, d_out), lambda i: (i, 0)),
      out_shape=jax.ShapeDtypeStruct((n, d_out), jnp.float32),
  )(y, p0, p1, b2d, wt)


def _relu_body(y_ref, p0_ref, p1_ref, b_ref, o_ref):
  o_ref[...] = jnp.maximum(
      y_ref[...] + p0_ref[...] + p1_ref[...] + b_ref[...], 0.0)


def _tc_fuse_relu(y, p0, p1, b2d):
  n, d = y.shape
  return pl.pallas_call(
      _relu_body,
      grid=(n // BLK,),
      in_specs=[
          pl.BlockSpec((BLK, d), lambda i: (i, 0)),
          pl.BlockSpec((BLK, d), lambda i: (i, 0)),
          pl.BlockSpec((BLK, d), lambda i: (i, 0)),
          pl.BlockSpec((1, d), lambda i: (0, 0)),
      ],
      out_specs=pl.BlockSpec((BLK, d), lambda i: (i, 0)),
      out_shape=jax.ShapeDtypeStruct((n, d), jnp.float32),
  )(y, p0, p1, b2d)


def _final_body(h_ref, p0_ref, p1_ref, w_ref, b_ref, o_ref):
  h = h_ref[...] + p0_ref[...] + p1_ref[...]
  o_ref[...] = jnp.dot(h, w_ref[...],
                       preferred_element_type=jnp.float32) + b_ref[...]


def _tc_final_matmul(h, p0, p1, wt, b2d):
  n, d = h.shape
  d_out = wt.shape[1]
  return pl.pallas_call(
      _final_body,
      grid=(n // BLK,),
      in_specs=[
          pl.BlockSpec((BLK, d), lambda i: (i, 0)),
          pl.BlockSpec((BLK, d), lambda i: (i, 0)),
          pl.BlockSpec((BLK, d), lambda i: (i, 0)),
          pl.BlockSpec((d, d_out), lambda i: (0, 0)),
          pl.BlockSpec((1, d_out), lambda i: (0, 0)),
      ],
      out_specs=pl.BlockSpec((BLK, d_out), lambda i: (i, 0)),
      out_shape=jax.ShapeDtypeStruct((n, d_out), jnp.float32),
  )(h, p0, p1, wt, b2d)


def kernel(x, edge_index, W1, b1, W2, b2, W3, b3):
  src = edge_index[0].astype(jnp.int32)
  dst = edge_index[1].astype(jnp.int32)
  pad = E_PAD - N_EDGES
  packed = jnp.bitwise_or(jnp.left_shift(src, 16), dst)
  packed = jnp.concatenate([packed, jnp.full((pad,), TRASH, jnp.int32)])
  edges_p = jnp.concatenate(
      [packed.reshape(NW, NROW, SLABW),
       jnp.zeros((NW, 1, SLABW), jnp.int32)], axis=1)

  w3p = jnp.zeros((D3, D_HID), jnp.float32).at[:2].set(W3)
  b3p = jnp.zeros((1, D3), jnp.float32).at[0, :2].set(b3)

  y1 = _tc_matmul(x, W1.T)
  p1a, p1b = _AGG128(y1, edges_p)
  y2 = _tc_fuse_matmul(y1, p1a[:N_NODES], p1b[:N_NODES], b1.reshape(1, -1),
                       W2.T)
  p2a, p2b = _AGG128(y2, edges_p)
  h2 = _tc_fuse_relu(y2, p2a[:N_NODES], p2b[:N_NODES], b2.reshape(1, -1))
  p3a, p3b = _AGG128(h2, edges_p)
  out = _tc_final_matmul(h2, p3a[:N_NODES], p3b[:N_NODES], w3p.T, b3p)
  return out[:, :2]


# speedup vs baseline: 1.3697x; 1.2096x over previous
import functools

import jax
import jax.numpy as jnp
from jax import lax
from jax.experimental import pallas as pl
from jax.experimental.pallas import tpu as pltpu
from jax.experimental.pallas import tpu_sc as plsc

N_NODES = 10000
N_EDGES = 320000
D_HID = 128
D3 = 16

NUM_CORES = 2
NUM_SUBCORES = 16
NW = NUM_CORES * NUM_SUBCORES
SLABW = 128
NROW = -(-N_EDGES // (NW * SLABW))
NROW = -(-NROW // 8) * 8
E_PAD = NW * NROW * SLABW
ROWS_PAD = 10240
STRIPE = ROWS_PAD // NUM_SUBCORES
TRASH = ROWS_PAD - 1


def _make_sc_agg(D):
  mesh = plsc.VectorSubcoreMesh(core_axis_name="c", subcore_axis_name="s")

  @functools.partial(
      pl.kernel,
      mesh=mesh,
      out_type=(
          jax.ShapeDtypeStruct((ROWS_PAD, D), jnp.float32),
          jax.ShapeDtypeStruct((ROWS_PAD, D), jnp.float32),
      ),
      scratch_types=[
          pltpu.VMEM_SHARED((ROWS_PAD, D), jnp.float32),
          pltpu.VMEM((NROW + 1, SLABW), jnp.int32),
          pltpu.VMEM((NROW + 1, SLABW), jnp.int32),
          pltpu.VMEM((16, D), jnp.float32),
          pltpu.VMEM((16, D), jnp.float32),
          pltpu.VMEM((16, D), jnp.float32),
          pltpu.SemaphoreType.DMA,
          pltpu.SemaphoreType.DMA,
      ],
  )
  def agg(y_hbm, src_hbm, dst_hbm, out0, out1, acc, src_v, dst_v, g0, g1, zb,
          sem0, sem1):
    cid = lax.axis_index("c")
    sid = lax.axis_index("s")
    w = cid * NUM_SUBCORES + sid

    zero = jnp.zeros((16,), jnp.float32)
    for r in range(16):
      for c in range(D // 16):
        zb[r, pl.ds(c * 16, 16)] = zero

    def zero_body(i, carry):
      pltpu.sync_copy(zb, acc.at[pl.ds(sid * STRIPE + i * 16, 16)])
      return carry

    lax.fori_loop(0, STRIPE // 16, zero_body, 0)

    pltpu.sync_copy(src_hbm.at[w], src_v)
    pltpu.sync_copy(dst_hbm.at[w], dst_v)

    plsc.subcore_barrier()

    bufs = (g0, g1)
    sems = (sem0, sem1)

    def src_of(j, u):
      return src_v[j, pl.ds(u * 16, 16)]

    def dst_of(j, u):
      return dst_v[j, pl.ds(u * 16, 16)]

    def gather_start(sv, buf, sem):
      pltpu.make_async_copy(y_hbm.at[sv], buf, sem).start()

    def gather_wait(sv, buf, sem):
      pltpu.make_async_copy(y_hbm.at[sv], buf, sem).wait()

    gather_start(src_of(0, 0), g0, sem0)

    def body(j, carry):
      for k in range(8):
        cur, nxt = bufs[k % 2], bufs[(k + 1) % 2]
        scur, snxt = sems[k % 2], sems[(k + 1) % 2]
        if k < 7:
          sv_n = src_of(j, k + 1)
        else:
          sv_n = src_of(j + 1, 0)
        gather_start(sv_n, nxt, snxt)
        gather_wait(src_of(j, k), cur, scur)
        pltpu.sync_copy(cur, acc.at[dst_of(j, k)], add=True)
      return carry

    lax.fori_loop(0, NROW, body, 0)

    gather_wait(src_of(NROW, 0), g0, sem0)

    plsc.subcore_barrier()

    @pl.when(cid == 0)
    def _():
      pltpu.sync_copy(acc.at[pl.ds(sid * STRIPE, STRIPE)],
                      out0.at[pl.ds(sid * STRIPE, STRIPE)])

    @pl.when(cid == 1)
    def _():
      pltpu.sync_copy(acc.at[pl.ds(sid * STRIPE, STRIPE)],
                      out1.at[pl.ds(sid * STRIPE, STRIPE)])

  return agg


_AGG128 = _make_sc_agg(D_HID)

BLK = 1000


def _mm_body(x_ref, w_ref, o_ref):
  o_ref[...] = jnp.dot(x_ref[...], w_ref[...],
                       preferred_element_type=jnp.float32)


def _tc_matmul(x, wt):
  n, k = x.shape
  d_out = wt.shape[1]
  return pl.pallas_call(
      _mm_body,
      grid=(n // BLK,),
      in_specs=[
          pl.BlockSpec((BLK, k), lambda i: (i, 0)),
          pl.BlockSpec((k, d_out), lambda i: (0, 0)),
      ],
      out_specs=pl.BlockSpec((BLK, d_out), lambda i: (i, 0)),
      out_shape=jax.ShapeDtypeStruct((n, d_out), jnp.float32),
  )(x, wt)


def _fuse_body(y_ref, p0_ref, p1_ref, b_ref, w_ref, o_ref):
  h = y_ref[...] + p0_ref[...] + p1_ref[...] + b_ref[...]
  h = jnp.maximum(h, 0.0)
  o_ref[...] = jnp.dot(h, w_ref[...], preferred_element_type=jnp.float32)


def _tc_fuse_matmul(y, p0, p1, b2d, wt):
  n, d = y.shape
  d_out = wt.shape[1]
  return pl.pallas_call(
      _fuse_body,
      grid=(n // BLK,),
      in_specs=[
          pl.BlockSpec((BLK, d), lambda i: (i, 0)),
          pl.BlockSpec((BLK, d), lambda i: (i, 0)),
          pl.BlockSpec((BLK, d), lambda i: (i, 0)),
          pl.BlockSpec((1, d), lambda i: (0, 0)),
          pl.BlockSpec((d, d_out), lambda i: (0, 0)),
      ],
      out_specs=pl.BlockSpec((BLK, d_out), lambda i: (i, 0)),
      out_shape=jax.ShapeDtypeStruct((n, d_out), jnp.float32),
  )(y, p0, p1, b2d, wt)


def _relu_body(y_ref, p0_ref, p1_ref, b_ref, o_ref):
  o_ref[...] = jnp.maximum(
      y_ref[...] + p0_ref[...] + p1_ref[...] + b_ref[...], 0.0)


def _tc_fuse_relu(y, p0, p1, b2d):
  n, d = y.shape
  return pl.pallas_call(
      _relu_body,
      grid=(n // BLK,),
      in_specs=[
          pl.BlockSpec((BLK, d), lambda i: (i, 0)),
          pl.BlockSpec((BLK, d), lambda i: (i, 0)),
          pl.BlockSpec((BLK, d), lambda i: (i, 0)),
          pl.BlockSpec((1, d), lambda i: (0, 0)),
      ],
      out_specs=pl.BlockSpec((BLK, d), lambda i: (i, 0)),
      out_shape=jax.ShapeDtypeStruct((n, d), jnp.float32),
  )(y, p0, p1, b2d)


def _final_body(h_ref, p0_ref, p1_ref, w_ref, b_ref, o_ref):
  h = h_ref[...] + p0_ref[...] + p1_ref[...]
  o_ref[...] = jnp.dot(h, w_ref[...],
                       preferred_element_type=jnp.float32) + b_ref[...]


def _tc_final_matmul(h, p0, p1, wt, b2d):
  n, d = h.shape
  d_out = wt.shape[1]
  return pl.pallas_call(
      _final_body,
      grid=(n // BLK,),
      in_specs=[
          pl.BlockSpec((BLK, d), lambda i: (i, 0)),
          pl.BlockSpec((BLK, d), lambda i: (i, 0)),
          pl.BlockSpec((BLK, d), lambda i: (i, 0)),
          pl.BlockSpec((d, d_out), lambda i: (0, 0)),
          pl.BlockSpec((1, d_out), lambda i: (0, 0)),
      ],
      out_specs=pl.BlockSpec((BLK, d_out), lambda i: (i, 0)),
      out_shape=jax.ShapeDtypeStruct((n, d_out), jnp.float32),
  )(h, p0, p1, wt, b2d)


def kernel(x, edge_index, W1, b1, W2, b2, W3, b3):
  src = edge_index[0].astype(jnp.int32)
  dst = edge_index[1].astype(jnp.int32)
  pad = E_PAD - N_EDGES
  zrow = jnp.zeros((NW, 1, SLABW), jnp.int32)

  def _slab(idx, fill):
    p = jnp.concatenate([idx, jnp.full((pad,), fill, jnp.int32)])
    return jnp.concatenate([p.reshape(NW, NROW, SLABW), zrow], axis=1)

  src_p = _slab(src, 0)
  dst_p = _slab(dst, TRASH)

  w3p = jnp.zeros((D3, D_HID), jnp.float32).at[:2].set(W3)
  b3p = jnp.zeros((1, D3), jnp.float32).at[0, :2].set(b3)

  y1 = _tc_matmul(x, W1.T)
  p1a, p1b = _AGG128(y1, src_p, dst_p)
  y2 = _tc_fuse_matmul(y1, p1a[:N_NODES], p1b[:N_NODES], b1.reshape(1, -1),
                       W2.T)
  p2a, p2b = _AGG128(y2, src_p, dst_p)
  h2 = _tc_fuse_relu(y2, p2a[:N_NODES], p2b[:N_NODES], b2.reshape(1, -1))
  p3a, p3b = _AGG128(h2, src_p, dst_p)
  out = _tc_final_matmul(h2, p3a[:N_NODES], p3b[:N_NODES], w3p.T, b3p)
  return out[:, :2]


# speedup vs baseline: 1.3697x; 1.0000x over previous
import functools

import jax
import jax.numpy as jnp
from jax import lax
from jax.experimental import pallas as pl
from jax.experimental.pallas import tpu as pltpu
from jax.experimental.pallas import tpu_sc as plsc

N_NODES = 10000
N_EDGES = 320000
D_HID = 128
D3 = 16

NUM_CORES = 2
NUM_SUBCORES = 16
NW = NUM_CORES * NUM_SUBCORES
SLABW = 128
NROW = -(-N_EDGES // (NW * SLABW))
NROW = -(-NROW // 8) * 8
E_PAD = NW * NROW * SLABW
ROWS_PAD = 10240
STRIPE = ROWS_PAD // NUM_SUBCORES
TRASH = ROWS_PAD - 1


def _make_sc_agg(D):
  mesh = plsc.VectorSubcoreMesh(core_axis_name="c", subcore_axis_name="s")

  @functools.partial(
      pl.kernel,
      mesh=mesh,
      out_type=(
          jax.ShapeDtypeStruct((ROWS_PAD, D), jnp.float32),
          jax.ShapeDtypeStruct((ROWS_PAD, D), jnp.float32),
      ),
      scratch_types=[
          pltpu.VMEM_SHARED((ROWS_PAD, D), jnp.float32),
          pltpu.VMEM((NROW + 1, SLABW), jnp.int32),
          pltpu.VMEM((NROW + 1, SLABW), jnp.int32),
          pltpu.VMEM((16, D), jnp.float32),
          pltpu.VMEM((16, D), jnp.float32),
          pltpu.VMEM((16, D), jnp.float32),
          pltpu.SemaphoreType.DMA,
          pltpu.SemaphoreType.DMA,
      ],
  )
  def agg(y_hbm, src_hbm, dst_hbm, out0, out1, acc, src_v, dst_v, g0, g1, zb,
          sem0, sem1):
    cid = lax.axis_index("c")
    sid = lax.axis_index("s")
    w = cid * NUM_SUBCORES + sid

    zero = jnp.zeros((16,), jnp.float32)
    for r in range(16):
      for c in range(D // 16):
        zb[r, pl.ds(c * 16, 16)] = zero

    def zero_body(i, carry):
      pltpu.sync_copy(zb, acc.at[pl.ds(sid * STRIPE + i * 16, 16)])
      return carry

    lax.fori_loop(0, STRIPE // 16, zero_body, 0)

    pltpu.sync_copy(src_hbm.at[w], src_v)
    pltpu.sync_copy(dst_hbm.at[w], dst_v)

    plsc.subcore_barrier()

    bufs = (g0, g1)
    sems = (sem0, sem1)

    def src_of(j, u):
      return src_v[j, pl.ds(u * 16, 16)]

    def dst_of(j, u):
      return dst_v[j, pl.ds(u * 16, 16)]

    def gather_start(sv, buf, sem):
      pltpu.make_async_copy(y_hbm.at[sv], buf, sem).start()

    def gather_wait(sv, buf, sem):
      pltpu.make_async_copy(y_hbm.at[sv], buf, sem).wait()

    sv0 = src_of(0, 0)
    gather_start(sv0, g0, sem0)

    def body(j, svc):
      for k in range(8):
        cur, nxt = bufs[k % 2], bufs[(k + 1) % 2]
        scur, snxt = sems[k % 2], sems[(k + 1) % 2]
        if k < 7:
          sv_n = src_of(j, k + 1)
        else:
          sv_n = src_of(j + 1, 0)
        gather_start(sv_n, nxt, snxt)
        gather_wait(svc, cur, scur)
        pltpu.sync_copy(cur, acc.at[dst_of(j, k)], add=True)
        svc = sv_n
      return svc

    svl = lax.fori_loop(0, NROW, body, sv0)

    gather_wait(svl, g0, sem0)

    plsc.subcore_barrier()

    @pl.when(cid == 0)
    def _():
      pltpu.sync_copy(acc.at[pl.ds(sid * STRIPE, STRIPE)],
                      out0.at[pl.ds(sid * STRIPE, STRIPE)])

    @pl.when(cid == 1)
    def _():
      pltpu.sync_copy(acc.at[pl.ds(sid * STRIPE, STRIPE)],
                      out1.at[pl.ds(sid * STRIPE, STRIPE)])

  return agg


_AGG128 = _make_sc_agg(D_HID)

BLK = 1000


def _mm_body(x_ref, w_ref, o_ref):
  o_ref[...] = jnp.dot(x_ref[...], w_ref[...],
                       preferred_element_type=jnp.float32)


def _tc_matmul(x, wt):
  n, k = x.shape
  d_out = wt.shape[1]
  return pl.pallas_call(
      _mm_body,
      grid=(n // BLK,),
      in_specs=[
          pl.BlockSpec((BLK, k), lambda i: (i, 0)),
          pl.BlockSpec((k, d_out), lambda i: (0, 0)),
      ],
      out_specs=pl.BlockSpec((BLK, d_out), lambda i: (i, 0)),
      out_shape=jax.ShapeDtypeStruct((n, d_out), jnp.float32),
  )(x, wt)


def _fuse_body(y_ref, p0_ref, p1_ref, b_ref, w_ref, o_ref):
  h = y_ref[...] + p0_ref[...] + p1_ref[...] + b_ref[...]
  h = jnp.maximum(h, 0.0)
  o_ref[...] = jnp.dot(h, w_ref[...], preferred_element_type=jnp.float32)


def _tc_fuse_matmul(y, p0, p1, b2d, wt):
  n, d = y.shape
  d_out = wt.shape[1]
  return pl.pallas_call(
      _fuse_body,
      grid=(n // BLK,),
      in_specs=[
          pl.BlockSpec((BLK, d), lambda i: (i, 0)),
          pl.BlockSpec((BLK, d), lambda i: (i, 0)),
          pl.BlockSpec((BLK, d), lambda i: (i, 0)),
          pl.BlockSpec((1, d), lambda i: (0, 0)),
          pl.BlockSpec((d, d_out), lambda i: (0, 0)),
      ],
      out_specs=pl.BlockSpec((BLK, d_out), lambda i: (i, 0)),
      out_shape=jax.ShapeDtypeStruct((n, d_out), jnp.float32),
  )(y, p0, p1, b2d, wt)


def _relu_body(y_ref, p0_ref, p1_ref, b_ref, o_ref):
  o_ref[...] = jnp.maximum(
      y_ref[...] + p0_ref[...] + p1_ref[...] + b_ref[...], 0.0)


def _tc_fuse_relu(y, p0, p1, b2d):
  n, d = y.shape
  return pl.pallas_call(
      _relu_body,
      grid=(n // BLK,),
      in_specs=[
          pl.BlockSpec((BLK, d), lambda i: (i, 0)),
          pl.BlockSpec((BLK, d), lambda i: (i, 0)),
          pl.BlockSpec((BLK, d), lambda i: (i, 0)),
          pl.BlockSpec((1, d), lambda i: (0, 0)),
      ],
      out_specs=pl.BlockSpec((BLK, d), lambda i: (i, 0)),
      out_shape=jax.ShapeDtypeStruct((n, d), jnp.float32),
  )(y, p0, p1, b2d)


def _final_body(h_ref, p0_ref, p1_ref, w_ref, b_ref, o_ref):
  h = h_ref[...] + p0_ref[...] + p1_ref[...]
  o_ref[...] = jnp.dot(h, w_ref[...],
                       preferred_element_type=jnp.float32) + b_ref[...]


def _tc_final_matmul(h, p0, p1, wt, b2d):
  n, d = h.shape
  d_out = wt.shape[1]
  return pl.pallas_call(
      _final_body,
      grid=(n // BLK,),
      in_specs=[
          pl.BlockSpec((BLK, d), lambda i: (i, 0)),
          pl.BlockSpec((BLK, d), lambda i: (i, 0)),
          pl.BlockSpec((BLK, d), lambda i: (i, 0)),
          pl.BlockSpec((d, d_out), lambda i: (0, 0)),
          pl.BlockSpec((1, d_out), lambda i: (0, 0)),
      ],
      out_specs=pl.BlockSpec((BLK, d_out), lambda i: (i, 0)),
      out_shape=jax.ShapeDtypeStruct((n, d_out), jnp.float32),
  )(h, p0, p1, wt, b2d)


def kernel(x, edge_index, W1, b1, W2, b2, W3, b3):
  src = edge_index[0].astype(jnp.int32)
  dst = edge_index[1].astype(jnp.int32)
  pad = E_PAD - N_EDGES
  zrow = jnp.zeros((NW, 1, SLABW), jnp.int32)

  def _slab(idx, fill):
    p = jnp.concatenate([idx, jnp.full((pad,), fill, jnp.int32)])
    return jnp.concatenate([p.reshape(NW, NROW, SLABW), zrow], axis=1)

  src_p = _slab(src, 0)
  dst_p = _slab(dst, TRASH)

  w3p = jnp.zeros((D3, D_HID), jnp.float32).at[:2].set(W3)
  b3p = jnp.zeros((1, D3), jnp.float32).at[0, :2].set(b3)

  y1 = _tc_matmul(x, W1.T)
  p1a, p1b = _AGG128(y1, src_p, dst_p)
  y2 = _tc_fuse_matmul(y1, p1a[:N_NODES], p1b[:N_NODES], b1.reshape(1, -1),
                       W2.T)
  p2a, p2b = _AGG128(y2, src_p, dst_p)
  h2 = _tc_fuse_relu(y2, p2a[:N_NODES], p2b[:N_NODES], b2.reshape(1, -1))
  p3a, p3b = _AGG128(h2, src_p, dst_p)
  out = _tc_final_matmul(h2, p3a[:N_NODES], p3b[:N_NODES], w3p.T, b3p)
  return out[:, :2]
